# Initial kernel scaffold; baseline (speedup 1.0000x reference)
import jax
import jax.numpy as jnp
from jax.experimental import pallas as pl


def kernel(x, edge_index, W1, b1, W2, b2):
    raise NotImplementedError("write your pallas kernel here")


# speedup vs baseline: 26.2861x; 26.2861x over previous
import functools

import jax
import jax.numpy as jnp
from jax import lax
from jax.experimental import pallas as pl
from jax.experimental.pallas import tpu as pltpu
from jax.experimental.pallas import tpu_sc as plsc

N = 10000
D = 128
E = 320000

NC, NS, L = 2, 16, 16
NW = NC * NS
N_PAD = 10112
RPT = N_PAD // NS
CHUNK = 64
CHUNKS = 160
SEC = 4
SCH = CHUNKS // SEC
EP = CHUNKS * CHUNK
E_PAD = EP * NW
RB = 632
GRID = N_PAD // RB

_mesh = plsc.VectorSubcoreMesh(core_axis_name="c", subcore_axis_name="s")
_sc_params = pltpu.CompilerParams(needs_layout_passes=False)


@functools.partial(
    pl.kernel,
    out_type=jax.ShapeDtypeStruct((NW, N_PAD), jnp.float32),
    mesh=_mesh,
    compiler_params=_sc_params,
    scratch_types=[
        pltpu.VMEM((2048,), jnp.int32),
        pltpu.VMEM((N_PAD,), jnp.float32),
    ],
)
def _deg_kernel(dst_hbm, out_hbm, dst_v, deg_v):
    w = lax.axis_index("c") * NS + lax.axis_index("s")

    def zero_body(i, carry):
        deg_v[pl.ds(i * L, L)] = jnp.zeros((L,), jnp.float32)
        return carry

    lax.fori_loop(0, N_PAD // L, zero_body, 0)
    ones = jnp.ones((L,), jnp.float32)

    def blk_body(k, carry):
        pltpu.sync_copy(dst_hbm.at[pl.ds(w * EP + k * 2048, 2048)], dst_v)

        def add_body(i, c2):
            plsc.addupdate_scatter(deg_v, [dst_v[pl.ds(i * L, L)]], ones)
            return c2

        lax.fori_loop(0, 2048 // L, add_body, 0)
        return carry

    lax.fori_loop(0, EP // 2048, blk_body, 0)
    pltpu.sync_copy(deg_v, out_hbm.at[w])


@functools.partial(
    pl.kernel,
    out_type=jax.ShapeDtypeStruct((NC, N_PAD, D), jnp.float32),
    mesh=_mesh,
    compiler_params=_sc_params,
    scratch_types=[
        pltpu.VMEM((SCH, CHUNK), jnp.int32),
        pltpu.VMEM((SCH, CHUNK), jnp.int32),
        pltpu.VMEM((2, CHUNK, D), jnp.float32),
        pltpu.VMEM_SHARED((N_PAD, D), jnp.float32),
        pltpu.SemaphoreType.DMA,
        pltpu.SemaphoreType.DMA,
    ],
)
def _scat_kernel(g_hbm, src_hbm, dst_hbm, out_hbm,
                 src_v, dst_v, rows_v, acc_sh, sem0, sem1):
    c = lax.axis_index("c")
    s = lax.axis_index("s")
    w = c * NS + s
    stripe = pl.ds(s * RPT, RPT)
    pltpu.sync_copy(g_hbm.at[stripe], acc_sh.at[stripe])
    plsc.subcore_barrier()

    def gat(i, buf, sem):
        return pltpu.make_async_copy(g_hbm.at[src_v.at[i]], rows_v.at[buf], sem)

    npairs = SCH // 2

    def sec_body(k, carry):
        pltpu.sync_copy(src_hbm.at[w, pl.ds(k * SCH, SCH)], src_v)
        pltpu.sync_copy(dst_hbm.at[w, pl.ds(k * SCH, SCH)], dst_v)
        gat(0, 0, sem0).start()

        def body(j, carry2):
            i0 = j * 2
            gat(i0 + 1, 1, sem1).start()
            gat(i0, 0, sem0).wait()
            pltpu.sync_copy(rows_v.at[0], acc_sh.at[dst_v.at[i0]], add=True)

            @pl.when(j < npairs - 1)
            def _():
                gat(i0 + 2, 0, sem0).start()

            gat(i0 + 1, 1, sem1).wait()
            pltpu.sync_copy(rows_v.at[1], acc_sh.at[dst_v.at[i0 + 1]], add=True)
            return carry2

        lax.fori_loop(0, npairs, body, 0)
        return carry

    lax.fori_loop(0, SEC, sec_body, 0)
    plsc.subcore_barrier()
    pltpu.sync_copy(acc_sh.at[stripe], out_hbm.at[c, stripe])


def _dis_body(deg_ref, o_ref):
    o_ref[...] = lax.rsqrt(jnp.sum(deg_ref[...], axis=0) + 1.0)


_dis_call = pl.pallas_call(
    _dis_body,
    out_shape=jax.ShapeDtypeStruct((N_PAD,), jnp.float32),
)


def _mm1_body(x_ref, w_ref, dis_ref, o_ref):
    o_ref[...] = jnp.dot(x_ref[...], w_ref[...],
                         preferred_element_type=jnp.float32) * dis_ref[...]


_mm1_call = pl.pallas_call(
    _mm1_body,
    grid=(GRID,),
    in_specs=[
        pl.BlockSpec((RB, D), lambda i: (i, 0)),
        pl.BlockSpec((D, D), lambda i: (0, 0)),
        pl.BlockSpec((RB, 1), lambda i: (i, 0)),
    ],
    out_specs=pl.BlockSpec((RB, D), lambda i: (i, 0)),
    out_shape=jax.ShapeDtypeStruct((N_PAD, D), jnp.float32),
)


def _mm2_body(a_ref, g_ref, dis_ref, b_ref, w_ref, o_ref):
    t = (a_ref[0] + a_ref[1] - g_ref[...]) * dis_ref[...] + b_ref[...]
    t = jnp.maximum(t, 0.0)
    o_ref[...] = jnp.dot(t, w_ref[...],
                         preferred_element_type=jnp.float32) * dis_ref[...]


_mm2_call = pl.pallas_call(
    _mm2_body,
    grid=(GRID,),
    in_specs=[
        pl.BlockSpec((NC, RB, D), lambda i: (0, i, 0)),
        pl.BlockSpec((RB, D), lambda i: (i, 0)),
        pl.BlockSpec((RB, 1), lambda i: (i, 0)),
        pl.BlockSpec((1, D), lambda i: (0, 0)),
        pl.BlockSpec((D, D), lambda i: (0, 0)),
    ],
    out_specs=pl.BlockSpec((RB, D), lambda i: (i, 0)),
    out_shape=jax.ShapeDtypeStruct((N_PAD, D), jnp.float32),
)


def _fin_body(a_ref, g_ref, dis_ref, b_ref, o_ref):
    o_ref[...] = (a_ref[0] + a_ref[1] - g_ref[...]) * dis_ref[...] + b_ref[...]


_fin_call = pl.pallas_call(
    _fin_body,
    grid=(GRID,),
    in_specs=[
        pl.BlockSpec((NC, RB, D), lambda i: (0, i, 0)),
        pl.BlockSpec((RB, D), lambda i: (i, 0)),
        pl.BlockSpec((RB, 1), lambda i: (i, 0)),
        pl.BlockSpec((1, D), lambda i: (0, 0)),
    ],
    out_specs=pl.BlockSpec((RB, D), lambda i: (i, 0)),
    out_shape=jax.ShapeDtypeStruct((N_PAD, D), jnp.float32),
)


def kernel(x, edge_index, W1, b1, W2, b2):
    src = edge_index[0].astype(jnp.int32)
    dst = edge_index[1].astype(jnp.int32)
    pad = N + (jnp.arange(E_PAD - E, dtype=jnp.int32) % (N_PAD - N))
    src_flat = jnp.concatenate([src, pad])
    dst_flat = jnp.concatenate([dst, pad])
    src3 = src_flat.reshape(NW, CHUNKS, CHUNK)
    dst3 = dst_flat.reshape(NW, CHUNKS, CHUNK)
    x_pad = jnp.concatenate(
        [x.astype(jnp.float32), jnp.zeros((N_PAD - N, D), jnp.float32)])

    deg_parts = _deg_kernel(dst_flat)
    dis = _dis_call(deg_parts)
    dis_col = dis.reshape(N_PAD, 1)

    g1 = _mm1_call(x_pad, W1, dis_col)
    acc1 = _scat_kernel(g1, src3, dst3)
    g2 = _mm2_call(acc1, g1, dis_col, b1.reshape(1, D), W2)
    acc2 = _scat_kernel(g2, src3, dst3)
    out = _fin_call(acc2, g2, dis_col, b2.reshape(1, D))
    return out[:N]


# speedup vs baseline: 29.6189x; 1.1268x over previous
import functools

import jax
import jax.numpy as jnp
from jax import lax
from jax.experimental import pallas as pl
from jax.experimental.pallas import tpu as pltpu
from jax.experimental.pallas import tpu_sc as plsc

N = 10000
D = 128
E = 320000

NC, NS, L = 2, 16, 16
NW = NC * NS
N_PAD = 10112
RPT = N_PAD // NS
CHUNK = 128
CHUNKS = 80
SEC = 4
SCH = CHUNKS // SEC
EP = CHUNKS * CHUNK
E_PAD = EP * NW
RB = 632
GRID = N_PAD // RB

_mesh = plsc.VectorSubcoreMesh(core_axis_name="c", subcore_axis_name="s")
_sc_params = pltpu.CompilerParams(needs_layout_passes=False)


@functools.partial(
    pl.kernel,
    out_type=jax.ShapeDtypeStruct((NW, N_PAD), jnp.float32),
    mesh=_mesh,
    compiler_params=_sc_params,
    scratch_types=[
        pltpu.VMEM((2048,), jnp.int32),
        pltpu.VMEM((N_PAD,), jnp.float32),
    ],
)
def _deg_kernel(dst_hbm, out_hbm, dst_v, deg_v):
    w = lax.axis_index("c") * NS + lax.axis_index("s")

    def zero_body(i, carry):
        deg_v[pl.ds(i * L, L)] = jnp.zeros((L,), jnp.float32)
        return carry

    lax.fori_loop(0, N_PAD // L, zero_body, 0)
    ones = jnp.ones((L,), jnp.float32)

    def blk_body(k, carry):
        pltpu.sync_copy(dst_hbm.at[pl.ds(w * EP + k * 2048, 2048)], dst_v)

        def add_body(i, c2):
            plsc.addupdate_scatter(deg_v, [dst_v[pl.ds(i * L, L)]], ones)
            return c2

        lax.fori_loop(0, 2048 // L, add_body, 0)
        return carry

    lax.fori_loop(0, EP // 2048, blk_body, 0)
    pltpu.sync_copy(deg_v, out_hbm.at[w])


@functools.partial(
    pl.kernel,
    out_type=jax.ShapeDtypeStruct((NC, N_PAD, D), jnp.float32),
    mesh=_mesh,
    compiler_params=_sc_params,
    scratch_types=[
        pltpu.VMEM((SCH, CHUNK), jnp.int32),
        pltpu.VMEM((SCH, CHUNK), jnp.int32),
        pltpu.VMEM((2, CHUNK, D), jnp.float32),
        pltpu.VMEM_SHARED((N_PAD, D), jnp.float32),
        pltpu.SemaphoreType.DMA,
        pltpu.SemaphoreType.DMA,
    ],
)
def _scat_kernel(g_hbm, src_hbm, dst_hbm, out_hbm,
                 src_v, dst_v, rows_v, acc_sh, sem0, sem1):
    c = lax.axis_index("c")
    s = lax.axis_index("s")
    w = c * NS + s
    stripe = pl.ds(s * RPT, RPT)
    pltpu.sync_copy(g_hbm.at[stripe], acc_sh.at[stripe])
    plsc.subcore_barrier()

    def gat(i, buf, sem):
        return pltpu.make_async_copy(g_hbm.at[src_v.at[i]], rows_v.at[buf], sem)

    npairs = SCH // 2

    def sec_body(k, carry):
        pltpu.sync_copy(src_hbm.at[k, w], src_v)
        pltpu.sync_copy(dst_hbm.at[k, w], dst_v)
        gat(0, 0, sem0).start()

        def body(j, carry2):
            i0 = j * 2
            gat(i0 + 1, 1, sem1).start()
            gat(i0, 0, sem0).wait()
            pltpu.sync_copy(rows_v.at[0], acc_sh.at[dst_v.at[i0]], add=True)

            @pl.when(j < npairs - 1)
            def _():
                gat(i0 + 2, 0, sem0).start()

            gat(i0 + 1, 1, sem1).wait()
            pltpu.sync_copy(rows_v.at[1], acc_sh.at[dst_v.at[i0 + 1]], add=True)
            return carry2

        lax.fori_loop(0, npairs, body, 0)
        return carry

    lax.fori_loop(0, SEC, sec_body, 0)
    plsc.subcore_barrier()
    pltpu.sync_copy(acc_sh.at[stripe], out_hbm.at[c, stripe])


def _dis_body(deg_ref, o_ref):
    o_ref[...] = lax.rsqrt(jnp.sum(deg_ref[...], axis=0) + 1.0)


_dis_call = pl.pallas_call(
    _dis_body,
    out_shape=jax.ShapeDtypeStruct((N_PAD,), jnp.float32),
)


def _mm1_body(x_ref, w_ref, dis_ref, o_ref):
    o_ref[...] = jnp.dot(x_ref[...], w_ref[...],
                         preferred_element_type=jnp.float32) * dis_ref[...]


_mm1_call = pl.pallas_call(
    _mm1_body,
    grid=(GRID,),
    in_specs=[
        pl.BlockSpec((RB, D), lambda i: (i, 0)),
        pl.BlockSpec((D, D), lambda i: (0, 0)),
        pl.BlockSpec((RB, 1), lambda i: (i, 0)),
    ],
    out_specs=pl.BlockSpec((RB, D), lambda i: (i, 0)),
    out_shape=jax.ShapeDtypeStruct((N_PAD, D), jnp.float32),
)


def _mm2_body(a_ref, g_ref, dis_ref, b_ref, w_ref, o_ref):
    t = (a_ref[0] + a_ref[1] - g_ref[...]) * dis_ref[...] + b_ref[...]
    t = jnp.maximum(t, 0.0)
    o_ref[...] = jnp.dot(t, w_ref[...],
                         preferred_element_type=jnp.float32) * dis_ref[...]


_mm2_call = pl.pallas_call(
    _mm2_body,
    grid=(GRID,),
    in_specs=[
        pl.BlockSpec((NC, RB, D), lambda i: (0, i, 0)),
        pl.BlockSpec((RB, D), lambda i: (i, 0)),
        pl.BlockSpec((RB, 1), lambda i: (i, 0)),
        pl.BlockSpec((1, D), lambda i: (0, 0)),
        pl.BlockSpec((D, D), lambda i: (0, 0)),
    ],
    out_specs=pl.BlockSpec((RB, D), lambda i: (i, 0)),
    out_shape=jax.ShapeDtypeStruct((N_PAD, D), jnp.float32),
)


def _fin_body(a_ref, g_ref, dis_ref, b_ref, o_ref):
    o_ref[...] = (a_ref[0] + a_ref[1] - g_ref[...]) * dis_ref[...] + b_ref[...]


_fin_call = pl.pallas_call(
    _fin_body,
    grid=(GRID,),
    in_specs=[
        pl.BlockSpec((NC, RB, D), lambda i: (0, i, 0)),
        pl.BlockSpec((RB, D), lambda i: (i, 0)),
        pl.BlockSpec((RB, 1), lambda i: (i, 0)),
        pl.BlockSpec((1, D), lambda i: (0, 0)),
    ],
    out_specs=pl.BlockSpec((RB, D), lambda i: (i, 0)),
    out_shape=jax.ShapeDtypeStruct((N_PAD, D), jnp.float32),
)


def kernel(x, edge_index, W1, b1, W2, b2):
    src = edge_index[0].astype(jnp.int32)
    dst = edge_index[1].astype(jnp.int32)
    pad = N + (jnp.arange(E_PAD - E, dtype=jnp.int32) % (N_PAD - N))
    src_flat = jnp.concatenate([src, pad])
    dst_flat = jnp.concatenate([dst, pad])
    src3 = src_flat.reshape(NW, SEC, SCH, CHUNK).transpose(1, 0, 2, 3)
    dst3 = dst_flat.reshape(NW, SEC, SCH, CHUNK).transpose(1, 0, 2, 3)
    x_pad = jnp.concatenate(
        [x.astype(jnp.float32), jnp.zeros((N_PAD - N, D), jnp.float32)])

    deg_parts = _deg_kernel(dst_flat)
    dis = _dis_call(deg_parts)
    dis_col = dis.reshape(N_PAD, 1)

    g1 = _mm1_call(x_pad, W1, dis_col)
    acc1 = _scat_kernel(g1, src3, dst3)
    g2 = _mm2_call(acc1, g1, dis_col, b1.reshape(1, D), W2)
    acc2 = _scat_kernel(g2, src3, dst3)
    out = _fin_call(acc2, g2, dis_col, b2.reshape(1, D))
    return out[:N]
